# baseline (speedup 1.0000x reference)
import functools

import jax
import jax.numpy as jnp
from jax import lax
from jax.experimental import pallas as pl
from jax.experimental.pallas import tpu as pltpu
from jax.experimental.pallas import tpu_sc as plsc


@functools.lru_cache(maxsize=None)
def _make_sc_gather(B, SP, V, DP, NBUF):
    info = plsc.get_sparse_core_info()
    NC, NS = info.num_cores, info.num_subcores
    NW = NC * NS
    assert B % NW == 0 and SP % 8 == 0 and DP % 128 == 0
    b_per_w = B // NW
    assert b_per_w % NBUF == 0 and b_per_w >= NBUF >= 2
    mesh = plsc.VectorSubcoreMesh(core_axis_name="c", subcore_axis_name="s")

    @functools.partial(
        pl.kernel,
        mesh=mesh,
        compiler_params=pltpu.CompilerParams(use_tc_tiling_on_sc=True),
        out_type=jax.ShapeDtypeStruct((B, SP, DP), jnp.float32),
        scratch_types=(
            [pltpu.VMEM((b_per_w * SP,), jnp.int32)]
            + [pltpu.VMEM((SP, DP), jnp.float32) for _ in range(NBUF)]
            + [pltpu.SemaphoreType.DMA for _ in range(2 * NBUF)]
        ),
    )
    def gather_kernel(table_hbm, idx_hbm, out_hbm, idx_v, *rest):
        bufs = rest[:NBUF]
        gsems = rest[NBUF:2 * NBUF]
        ssems = rest[2 * NBUF:3 * NBUF]
        wid = lax.axis_index("s") * NC + lax.axis_index("c")
        base = wid * b_per_w
        pltpu.sync_copy(idx_hbm.at[pl.ds(base * SP, b_per_w * SP)], idx_v)

        def start_gather(k, s):
            pltpu.async_copy(
                table_hbm.at[idx_v.at[pl.ds(k * SP, SP)]], bufs[s], gsems[s])

        def wait_gather(s):
            pltpu.make_async_copy(
                table_hbm.at[idx_v.at[pl.ds(0, SP)]], bufs[s], gsems[s]).wait()

        def start_scatter(k, s):
            pltpu.async_copy(bufs[s], out_hbm.at[base + k], ssems[s])

        def wait_scatter(s):
            pltpu.make_async_copy(bufs[s], out_hbm.at[base], ssems[s]).wait()

        for j in range(NBUF - 1):
            start_gather(j, j)

        def group_body(g, carry):
            for b in range(NBUF):
                k = g * NBUF + b
                pb = (b - 1) % NBUF

                @pl.when(k + NBUF - 1 < b_per_w)
                def _():
                    @pl.when(k >= 1)
                    def _():
                        wait_scatter(pb)

                    start_gather(k + NBUF - 1, pb)

                wait_gather(b)
                start_scatter(k, b)
            return carry

        lax.fori_loop(0, b_per_w // NBUF, group_body, 0)
        for s in range(NBUF):
            wait_scatter(s)

    return gather_kernel


@functools.lru_cache(maxsize=None)
def _make_tc_strip(B, S, SP, D, DP, BB):
    assert B % BB == 0

    def strip_body(y_ref, o_ref):
        o_ref[...] = y_ref[:, :S, :D]

    return pl.pallas_call(
        strip_body,
        grid=(B // BB,),
        in_specs=[pl.BlockSpec((BB, SP, DP), lambda b: (b, 0, 0))],
        out_specs=pl.BlockSpec((BB, S, D), lambda b: (b, 0, 0)),
        out_shape=jax.ShapeDtypeStruct((B, S, D), jnp.float32),
    )


def kernel(contexts, table):
    B, S = contexts.shape
    V, D = table.shape
    SP = (S + 7) // 8 * 8
    DP = (D + 127) // 128 * 128
    idx = jnp.pad(contexts.astype(jnp.int32), ((0, 0), (0, SP - S)))
    table_pad = jnp.pad(table, ((0, 0), (0, DP - D)))
    y = _make_sc_gather(B, SP, V, DP, 2)(table_pad, idx.reshape(B * SP))
    return _make_tc_strip(B, S, SP, D, DP, 4)(y)


# speedup vs baseline: 1.0885x; 1.0885x over previous
import functools

import jax
import jax.numpy as jnp
from jax import lax
from jax.experimental import pallas as pl
from jax.experimental.pallas import tpu as pltpu
from jax.experimental.pallas import tpu_sc as plsc


@functools.lru_cache(maxsize=None)
def _make_sc_gather(B, SP, V, DP, NBUF):
    info = plsc.get_sparse_core_info()
    NC, NS = info.num_cores, info.num_subcores
    NW = NC * NS
    assert B % NW == 0 and SP % 8 == 0
    b_per_w = B // NW
    assert b_per_w % NBUF == 0 and b_per_w >= NBUF >= 2
    mesh = plsc.VectorSubcoreMesh(core_axis_name="c", subcore_axis_name="s")

    @functools.partial(
        pl.kernel,
        mesh=mesh,
        compiler_params=pltpu.CompilerParams(use_tc_tiling_on_sc=False),
        out_type=jax.ShapeDtypeStruct((B, SP, DP), jnp.float32),
        scratch_types=(
            [pltpu.VMEM((b_per_w * SP,), jnp.int32)]
            + [pltpu.VMEM((SP, DP), jnp.float32) for _ in range(NBUF)]
            + [pltpu.SemaphoreType.DMA for _ in range(2 * NBUF)]
        ),
    )
    def gather_kernel(table_hbm, idx_hbm, out_hbm, idx_v, *rest):
        bufs = rest[:NBUF]
        gsems = rest[NBUF:2 * NBUF]
        ssems = rest[2 * NBUF:3 * NBUF]
        wid = lax.axis_index("s") * NC + lax.axis_index("c")
        base = wid * b_per_w
        pltpu.sync_copy(idx_hbm.at[pl.ds(base * SP, b_per_w * SP)], idx_v)

        def start_gather(k, s):
            pltpu.async_copy(
                table_hbm.at[idx_v.at[pl.ds(k * SP, SP)]], bufs[s], gsems[s])

        def wait_gather(s):
            pltpu.make_async_copy(
                table_hbm.at[idx_v.at[pl.ds(0, SP)]], bufs[s], gsems[s]).wait()

        def start_scatter(k, s):
            pltpu.async_copy(bufs[s], out_hbm.at[base + k], ssems[s])

        def wait_scatter(s):
            pltpu.make_async_copy(bufs[s], out_hbm.at[base], ssems[s]).wait()

        for j in range(NBUF - 1):
            start_gather(j, j)

        def group_body(g, carry):
            for b in range(NBUF):
                k = g * NBUF + b
                pb = (b - 1) % NBUF

                @pl.when(k + NBUF - 1 < b_per_w)
                def _():
                    @pl.when(k >= 1)
                    def _():
                        wait_scatter(pb)

                    start_gather(k + NBUF - 1, pb)

                wait_gather(b)
                start_scatter(k, b)
            return carry

        lax.fori_loop(0, b_per_w // NBUF, group_body, 0)
        for s in range(NBUF):
            wait_scatter(s)

    return gather_kernel


@functools.lru_cache(maxsize=None)
def _make_tc_strip(B, S, SP, D, DP, BB):
    assert B % BB == 0

    def strip_body(y_ref, o_ref):
        o_ref[...] = y_ref[:, :S, :D]

    return pl.pallas_call(
        strip_body,
        grid=(B // BB,),
        in_specs=[pl.BlockSpec((BB, SP, DP), lambda b: (b, 0, 0))],
        out_specs=pl.BlockSpec((BB, S, D), lambda b: (b, 0, 0)),
        out_shape=jax.ShapeDtypeStruct((B, S, D), jnp.float32),
    )


def kernel(contexts, table):
    B, S = contexts.shape
    V, D = table.shape
    SP = (S + 7) // 8 * 8
    DP = (D + 127) // 128 * 128
    idx = jnp.pad(contexts.astype(jnp.int32), ((0, 0), (0, SP - S)))
    y = _make_sc_gather(B, SP, V, D, 2)(table, idx.reshape(B * SP))
    return y[:, :S, :]


# speedup vs baseline: 1.1002x; 1.0107x over previous
import functools

import jax
import jax.numpy as jnp
from jax import lax
from jax.experimental import pallas as pl
from jax.experimental.pallas import tpu as pltpu
from jax.experimental.pallas import tpu_sc as plsc


@functools.lru_cache(maxsize=None)
def _make_sc_gather(N, V, D, C, NBUF):
    info = plsc.get_sparse_core_info()
    NC, NS = info.num_cores, info.num_subcores
    NW = NC * NS
    assert N % NW == 0
    n_per_w = N // NW
    assert n_per_w % C == 0 and C % 8 == 0
    n_chunks = n_per_w // C
    assert n_chunks % NBUF == 0 and n_chunks >= NBUF >= 2
    mesh = plsc.VectorSubcoreMesh(core_axis_name="c", subcore_axis_name="s")

    @functools.partial(
        pl.kernel,
        mesh=mesh,
        compiler_params=pltpu.CompilerParams(use_tc_tiling_on_sc=False),
        out_type=jax.ShapeDtypeStruct((N, D), jnp.float32),
        scratch_types=(
            [pltpu.VMEM((n_per_w,), jnp.int32)]
            + [pltpu.VMEM((C, D), jnp.float32) for _ in range(NBUF)]
            + [pltpu.SemaphoreType.DMA for _ in range(2 * NBUF)]
        ),
    )
    def gather_kernel(table_hbm, idx_hbm, out_hbm, idx_v, *rest):
        bufs = rest[:NBUF]
        gsems = rest[NBUF:2 * NBUF]
        ssems = rest[2 * NBUF:3 * NBUF]
        wid = lax.axis_index("s") * NC + lax.axis_index("c")
        base = wid * n_per_w
        pltpu.sync_copy(idx_hbm.at[pl.ds(base, n_per_w)], idx_v)

        def start_gather(i, s):
            pltpu.async_copy(
                table_hbm.at[idx_v.at[pl.ds(i * C, C)]], bufs[s], gsems[s])

        def wait_gather(s):
            pltpu.make_async_copy(
                table_hbm.at[idx_v.at[pl.ds(0, C)]], bufs[s], gsems[s]).wait()

        def start_scatter(i, s):
            pltpu.async_copy(
                bufs[s], out_hbm.at[pl.ds(base + i * C, C)], ssems[s])

        def wait_scatter(s):
            pltpu.make_async_copy(
                bufs[s], out_hbm.at[pl.ds(base, C)], ssems[s]).wait()

        for j in range(NBUF - 1):
            start_gather(j, j)

        def group_body(g, carry):
            for b in range(NBUF):
                i = g * NBUF + b
                pb = (b - 1) % NBUF

                @pl.when(i + NBUF - 1 < n_chunks)
                def _():
                    @pl.when(i >= 1)
                    def _():
                        wait_scatter(pb)

                    start_gather(i + NBUF - 1, pb)

                wait_gather(b)
                start_scatter(i, b)
            return carry

        lax.fori_loop(0, n_chunks // NBUF, group_body, 0)
        for s in range(NBUF):
            wait_scatter(s)

    return gather_kernel


def kernel(contexts, table):
    B, S = contexts.shape
    V, D = table.shape
    SP = (S + 7) // 8 * 8
    idx = jnp.pad(contexts.astype(jnp.int32), ((0, 0), (0, SP - S)))
    y = _make_sc_gather(B * SP, V, D, 16, 4)(table, idx.reshape(B * SP))
    return y.reshape(B, SP, D)[:, :S, :]


# speedup vs baseline: 1.8854x; 1.7137x over previous
import functools

import jax
import jax.numpy as jnp
from jax import lax
from jax.experimental import pallas as pl
from jax.experimental.pallas import tpu as pltpu
from jax.experimental.pallas import tpu_sc as plsc


@functools.lru_cache(maxsize=None)
def _make_sc_gather(N, V, D, C, NBUF):
    info = plsc.get_sparse_core_info()
    NC, NS = info.num_cores, info.num_subcores
    NW = NC * NS
    assert N % NW == 0
    n_per_w = N // NW
    assert n_per_w % C == 0 and C % 8 == 0
    n_chunks = n_per_w // C
    assert n_chunks % NBUF == 0 and n_chunks >= NBUF >= 2
    mesh = plsc.VectorSubcoreMesh(core_axis_name="c", subcore_axis_name="s")

    @functools.partial(
        pl.kernel,
        mesh=mesh,
        compiler_params=pltpu.CompilerParams(use_tc_tiling_on_sc=False),
        out_type=jax.ShapeDtypeStruct((N, D), jnp.float32),
        scratch_types=(
            [pltpu.VMEM((n_per_w,), jnp.int32)]
            + [pltpu.VMEM((C, D), jnp.float32) for _ in range(NBUF)]
            + [pltpu.SemaphoreType.DMA for _ in range(2 * NBUF)]
        ),
    )
    def gather_kernel(table_hbm, idx_hbm, out_hbm, idx_v, *rest):
        bufs = rest[:NBUF]
        gsems = rest[NBUF:2 * NBUF]
        ssems = rest[2 * NBUF:3 * NBUF]
        wid = lax.axis_index("s") * NC + lax.axis_index("c")
        base = wid * n_per_w
        pltpu.sync_copy(idx_hbm.at[pl.ds(base, n_per_w)], idx_v)

        def start_gather(i, s):
            pltpu.async_copy(
                table_hbm.at[idx_v.at[pl.ds(i * C, C)]], bufs[s], gsems[s])

        def wait_gather(s):
            pltpu.make_async_copy(
                table_hbm.at[idx_v.at[pl.ds(0, C)]], bufs[s], gsems[s]).wait()

        def start_scatter(i, s):
            pltpu.async_copy(
                bufs[s], out_hbm.at[pl.ds(base + i * C, C)], ssems[s])

        def wait_scatter(s):
            pltpu.make_async_copy(
                bufs[s], out_hbm.at[pl.ds(base, C)], ssems[s]).wait()

        for j in range(NBUF - 1):
            start_gather(j, j)

        def group_body(g, carry):
            for b in range(NBUF):
                i = g * NBUF + b
                pb = (b - 1) % NBUF

                @pl.when(i + NBUF - 1 < n_chunks)
                def _():
                    @pl.when(i >= 1)
                    def _():
                        wait_scatter(pb)

                    start_gather(i + NBUF - 1, pb)

                wait_gather(b)
                start_scatter(i, b)
            return carry

        lax.fori_loop(0, n_chunks // NBUF, group_body, 0)
        for s in range(NBUF):
            wait_scatter(s)

    return gather_kernel


def kernel(contexts, table):
    B, S = contexts.shape
    V, D = table.shape
    SP = (S + 7) // 8 * 8
    fill = (jax.lax.broadcasted_iota(jnp.int32, (B, SP - S), 0)
            * (SP - S)
            + jax.lax.broadcasted_iota(jnp.int32, (B, SP - S), 1)) % V
    idx = jnp.concatenate([contexts.astype(jnp.int32), fill], axis=1)
    y = _make_sc_gather(B * SP, V, D, 16, 4)(table, idx.reshape(B * SP))
    return y.reshape(B, SP, D)[:, :S, :]
